# Initial kernel scaffold; baseline (speedup 1.0000x reference)
import functools

import jax
import jax.numpy as jnp
from jax import lax
from jax.experimental import pallas as pl
from jax.experimental.pallas import tpu as pltpu
from jax.experimental.pallas import tpu_sc as plsc

N = 100000
E = 1600000
NC = 2
NS = 16
NW = NC * NS
CH = 128
IB = 50
STEPS = 400
NBLK = STEPS // IB
E_PAD = NW * STEPS * CH
N_ACC = 100096
ACC_PT = N_ACC // NS


def _make_spmm(D):
    mesh = plsc.VectorSubcoreMesh(core_axis_name="c", subcore_axis_name="s")

    @functools.partial(
        pl.kernel,
        out_type=jax.ShapeDtypeStruct((NC, N_ACC, D), jnp.float32),
        mesh=mesh,
        scratch_types=[
            pltpu.VMEM((IB, CH), jnp.int32),
            pltpu.VMEM((IB, CH), jnp.int32),
            pltpu.VMEM((CH, D), jnp.float32),
            pltpu.VMEM_SHARED((N_ACC, D), jnp.float32),
            pltpu.SemaphoreType.DMA,
        ],
    )
    def spmm(table_hbm, row_hbm, col_hbm, zeros_hbm, out_hbm,
             rowbuf, colbuf, gbuf, acc, sem):
        c = lax.axis_index("c")
        s = lax.axis_index("s")
        w = c * NS + s
        pltpu.sync_copy(zeros_hbm.at[pl.ds(s * ACC_PT, ACC_PT)],
                        acc.at[pl.ds(s * ACC_PT, ACC_PT)])
        plsc.subcore_barrier()
        for blk in range(NBLK):
            pltpu.sync_copy(row_hbm.at[w, pl.ds(blk * IB, IB)], rowbuf)
            pltpu.sync_copy(col_hbm.at[w, pl.ds(blk * IB, IB)], colbuf)

            def body(i, carry):
                pltpu.async_copy(table_hbm.at[rowbuf.at[i]], gbuf, sem).wait()
                pltpu.sync_copy(gbuf, acc.at[colbuf.at[i]], add=True)
                return carry

            lax.fori_loop(0, IB, body, 0)
        plsc.subcore_barrier()
        pltpu.sync_copy(acc.at[pl.ds(s * ACC_PT, ACC_PT)],
                        out_hbm.at[c, pl.ds(s * ACC_PT, ACC_PT)])

    return spmm


def _make_deg():
    mesh = plsc.VectorSubcoreMesh(core_axis_name="c", subcore_axis_name="s")

    @functools.partial(
        pl.kernel,
        out_type=jax.ShapeDtypeStruct((NC, N_ACC), jnp.float32),
        mesh=mesh,
        scratch_types=[
            pltpu.VMEM((IB, CH), jnp.int32),
            pltpu.VMEM((CH,), jnp.float32),
            pltpu.VMEM_SHARED((N_ACC,), jnp.float32),
        ],
    )
    def deg(col_hbm, ones_hbm, zeros_hbm, out_hbm, colbuf, onesbuf, acc):
        c = lax.axis_index("c")
        s = lax.axis_index("s")
        w = c * NS + s
        pltpu.sync_copy(ones_hbm, onesbuf)
        pltpu.sync_copy(zeros_hbm.at[pl.ds(s * ACC_PT, ACC_PT)],
                        acc.at[pl.ds(s * ACC_PT, ACC_PT)])
        plsc.subcore_barrier()
        for blk in range(NBLK):
            pltpu.sync_copy(col_hbm.at[w, pl.ds(blk * IB, IB)], colbuf)

            def body(i, carry):
                pltpu.sync_copy(onesbuf, acc.at[colbuf.at[i]], add=True)
                return carry

            lax.fori_loop(0, IB, body, 0)
        plsc.subcore_barrier()
        pltpu.sync_copy(acc.at[pl.ds(s * ACC_PT, ACC_PT)],
                        out_hbm.at[c, pl.ds(s * ACC_PT, ACC_PT)])

    return deg


_B = 1000


def _p1_body(deg_ref, x_ref, dis_ref, g1_ref):
    deg = deg_ref[0, :, :] + deg_ref[1, :, :] + 1.0
    dis = lax.rsqrt(deg)
    dis_ref[...] = dis
    g1_ref[...] = x_ref[...] * dis


def _p2_body(s1_ref, g1_ref, dis_ref, w1_ref, b1_ref, w2_ref,
             g2a_ref, g2b_ref):
    dis = dis_ref[...]
    t = (s1_ref[0, :, :] + s1_ref[1, :, :] + g1_ref[...]) * dis
    h1 = jnp.dot(t, w1_ref[...], preferred_element_type=jnp.float32)
    h1 = jnp.maximum(h1 + b1_ref[...], 0.0)
    g2 = jnp.dot(h1, w2_ref[...], preferred_element_type=jnp.float32) * dis
    g2a_ref[...] = g2[:, :16]
    g2b_ref[...] = g2[:, 16:]


def _p3_body(s2a_ref, s2b_ref, g2a_ref, g2b_ref, dis_ref, b2_ref,
             wc_ref, bc_ref, out_ref):
    dis = dis_ref[...]
    ua = (s2a_ref[0, :, :] + s2a_ref[1, :, :] + g2a_ref[...])
    ub = (s2b_ref[0, :, :] + s2b_ref[1, :, :] + g2b_ref[...])
    u = jnp.concatenate([ua, ub], axis=1) * dis
    h2 = jnp.maximum(u + b2_ref[...], 0.0)
    logits = jnp.dot(h2, wc_ref[...], preferred_element_type=jnp.float32)
    logits = logits + bc_ref[...]
    m = jnp.max(logits, axis=1, keepdims=True)
    lse = jnp.log(jnp.sum(jnp.exp(logits - m), axis=1, keepdims=True)) + m
    out_ref[...] = logits - lse


def _row_spec(d):
    return pl.BlockSpec((_B, d), lambda i: (i, 0))


def _full_spec(shape):
    nd = len(shape)
    return pl.BlockSpec(shape, lambda i: (0,) * nd)


def _part_spec(d):
    return pl.BlockSpec((NC, _B, d), lambda i: (0, i, 0))


def kernel(x, edge_index, W1, b1, W2, b2, Wc, bc):
    row = edge_index[0].astype(jnp.int32)
    col = edge_index[1].astype(jnp.int32)
    pad = E_PAD - E
    rowp = jnp.concatenate([row, jnp.zeros((pad,), jnp.int32)])
    colp = jnp.concatenate([col, jnp.full((pad,), N, jnp.int32)])
    row3 = rowp.reshape(NW, STEPS, CH)
    col3 = colp.reshape(NW, STEPS, CH)

    x8 = jnp.pad(x, ((0, 0), (0, 1)))
    w1p = jnp.pad(W1, ((0, 1), (0, 0)))
    b1r = b1.reshape(1, 64)
    b2r = b2.reshape(1, 32)
    bcr = bc.reshape(1, 2)

    zeros1 = jnp.zeros((N_ACC,), jnp.float32)
    zeros8 = jnp.zeros((N_ACC, 8), jnp.float32)
    zeros16 = jnp.zeros((N_ACC, 16), jnp.float32)
    ones = jnp.ones((CH,), jnp.float32)

    deg_part = _make_deg()(col3, ones, zeros1)
    deg3 = deg_part[:, :N].reshape(NC, N, 1)

    grid = (N // _B,)
    dis, g1 = pl.pallas_call(
        _p1_body,
        grid=grid,
        in_specs=[_part_spec(1), _row_spec(8)],
        out_specs=[_row_spec(1), _row_spec(8)],
        out_shape=[jax.ShapeDtypeStruct((N, 1), jnp.float32),
                   jax.ShapeDtypeStruct((N, 8), jnp.float32)],
    )(deg3, x8)

    spmm8 = _make_spmm(8)
    spmm16 = _make_spmm(16)

    s1 = spmm8(g1, row3, col3, zeros8)[:, :N, :]

    g2a, g2b = pl.pallas_call(
        _p2_body,
        grid=grid,
        in_specs=[_part_spec(8), _row_spec(8), _row_spec(1),
                  _full_spec((8, 64)), _full_spec((1, 64)),
                  _full_spec((64, 32))],
        out_specs=[_row_spec(16), _row_spec(16)],
        out_shape=[jax.ShapeDtypeStruct((N, 16), jnp.float32),
                   jax.ShapeDtypeStruct((N, 16), jnp.float32)],
    )(s1, g1, dis, w1p, b1r, W2)

    s2a = spmm16(g2a, row3, col3, zeros16)[:, :N, :]
    s2b = spmm16(g2b, row3, col3, zeros16)[:, :N, :]

    out = pl.pallas_call(
        _p3_body,
        grid=grid,
        in_specs=[_part_spec(16), _part_spec(16), _row_spec(16),
                  _row_spec(16), _row_spec(1), _full_spec((1, 32)),
                  _full_spec((32, 2)), _full_spec((1, 2))],
        out_specs=_row_spec(2),
        out_shape=jax.ShapeDtypeStruct((N, 2), jnp.float32),
    )(s2a, s2b, g2a, g2b, dis, b2r, Wc, bcr)

    return out


# speedup vs baseline: 18.1688x; 18.1688x over previous
import functools

import jax
import jax.numpy as jnp
from jax import lax
from jax.experimental import pallas as pl
from jax.experimental.pallas import tpu as pltpu
from jax.experimental.pallas import tpu_sc as plsc

N = 100000
E = 1600000
NC = 2
NS = 16
NW = NC * NS
CH = 128
IB = 40
STEPS = 400
NBLK = STEPS // IB
E_PAD = NW * STEPS * CH
N_ACC = 100096
ACC_PT = N_ACC // NS
NACCD = 114688
ACC_PTD = NACCD // NS


def _make_spmm(D):
    mesh = plsc.VectorSubcoreMesh(core_axis_name="c", subcore_axis_name="s")

    @functools.partial(
        pl.kernel,
        out_type=jax.ShapeDtypeStruct((NC, N_ACC, D), jnp.float32),
        mesh=mesh,
        scratch_types=[
            pltpu.VMEM((IB, CH), jnp.int32),
            pltpu.VMEM((IB, CH), jnp.int32),
            pltpu.VMEM((CH, D), jnp.float32),
            pltpu.VMEM_SHARED((N_ACC, D), jnp.float32),
            pltpu.SemaphoreType.DMA,
        ],
        compiler_params=pltpu.CompilerParams(use_tc_tiling_on_sc=False),
    )
    def spmm(table_hbm, row_hbm, col_hbm, zeros_hbm, out_hbm,
             rowbuf, colbuf, gbuf, acc, sem):
        c = lax.axis_index("c")
        s = lax.axis_index("s")
        w = c * NS + s
        pltpu.sync_copy(zeros_hbm.at[pl.ds(s * ACC_PT, ACC_PT)],
                        acc.at[pl.ds(s * ACC_PT, ACC_PT)])
        plsc.subcore_barrier()
        for blk in range(NBLK):
            pltpu.sync_copy(row_hbm.at[w, pl.ds(blk * IB, IB)], rowbuf)
            pltpu.sync_copy(col_hbm.at[w, pl.ds(blk * IB, IB)], colbuf)

            def body(i, carry):
                pltpu.async_copy(table_hbm.at[rowbuf.at[i]], gbuf, sem).wait()
                pltpu.sync_copy(gbuf, acc.at[colbuf.at[i]], add=True)
                return carry

            lax.fori_loop(0, IB, body, 0)
        plsc.subcore_barrier()
        pltpu.sync_copy(acc.at[pl.ds(s * ACC_PT, ACC_PT)],
                        out_hbm.at[c, pl.ds(s * ACC_PT, ACC_PT)])

    return spmm


def _make_deg():
    mesh = plsc.VectorSubcoreMesh(core_axis_name="c", subcore_axis_name="s")

    @functools.partial(
        pl.kernel,
        out_type=jax.ShapeDtypeStruct((NC * NACCD,), jnp.float32),
        mesh=mesh,
        scratch_types=[
            pltpu.VMEM((IB, CH), jnp.int32),
            pltpu.VMEM((CH,), jnp.float32),
            pltpu.VMEM_SHARED((NACCD,), jnp.float32),
        ],
        compiler_params=pltpu.CompilerParams(use_tc_tiling_on_sc=False),
    )
    def deg(col_hbm, ones_hbm, zeros_hbm, out_hbm, colbuf, onesbuf, acc):
        c = lax.axis_index("c")
        s = lax.axis_index("s")
        w = c * NS + s
        pltpu.sync_copy(ones_hbm, onesbuf)
        pltpu.sync_copy(zeros_hbm.at[pl.ds(s * ACC_PTD, ACC_PTD)],
                        acc.at[pl.ds(s * ACC_PTD, ACC_PTD)])
        plsc.subcore_barrier()
        for blk in range(NBLK):
            pltpu.sync_copy(col_hbm.at[w, pl.ds(blk * IB, IB)], colbuf)

            def body(i, carry):
                pltpu.sync_copy(onesbuf, acc.at[colbuf.at[i]], add=True)
                return carry

            lax.fori_loop(0, IB, body, 0)
        plsc.subcore_barrier()
        pltpu.sync_copy(acc.at[pl.ds(s * ACC_PTD, ACC_PTD)],
                        out_hbm.at[pl.ds(c * NACCD + s * ACC_PTD, ACC_PTD)])

    return deg


_B = 1000


def _p1_body(deg_ref, x_ref, dis_ref, g1_ref):
    deg = deg_ref[0, :, :] + deg_ref[1, :, :] + 1.0
    dis = lax.rsqrt(deg)
    dis_ref[...] = dis
    g1_ref[...] = x_ref[...] * dis


def _p2_body(s1_ref, g1_ref, dis_ref, w1_ref, b1_ref, w2_ref,
             g2a_ref, g2b_ref):
    dis = dis_ref[...]
    t = (s1_ref[0, :, :] + s1_ref[1, :, :] + g1_ref[...]) * dis
    h1 = jnp.dot(t, w1_ref[...], preferred_element_type=jnp.float32)
    h1 = jnp.maximum(h1 + b1_ref[...], 0.0)
    g2 = jnp.dot(h1, w2_ref[...], preferred_element_type=jnp.float32) * dis
    g2a_ref[...] = g2[:, :16]
    g2b_ref[...] = g2[:, 16:]


def _p3_body(s2a_ref, s2b_ref, g2a_ref, g2b_ref, dis_ref, b2_ref,
             wc_ref, bc_ref, out_ref):
    dis = dis_ref[...]
    ua = (s2a_ref[0, :, :] + s2a_ref[1, :, :] + g2a_ref[...])
    ub = (s2b_ref[0, :, :] + s2b_ref[1, :, :] + g2b_ref[...])
    u = jnp.concatenate([ua, ub], axis=1) * dis
    h2 = jnp.maximum(u + b2_ref[...], 0.0)
    logits = jnp.dot(h2, wc_ref[...], preferred_element_type=jnp.float32)
    logits = logits + bc_ref[...]
    m = jnp.max(logits, axis=1, keepdims=True)
    lse = jnp.log(jnp.sum(jnp.exp(logits - m), axis=1, keepdims=True)) + m
    out_ref[...] = logits - lse


def _row_spec(d):
    return pl.BlockSpec((_B, d), lambda i: (i, 0))


def _full_spec(shape):
    nd = len(shape)
    return pl.BlockSpec(shape, lambda i: (0,) * nd)


def _part_spec(d):
    return pl.BlockSpec((NC, _B, d), lambda i: (0, i, 0))


def kernel(x, edge_index, W1, b1, W2, b2, Wc, bc):
    row = edge_index[0].astype(jnp.int32)
    col = edge_index[1].astype(jnp.int32)
    pad = E_PAD - E
    rowp = jnp.concatenate([row, jnp.zeros((pad,), jnp.int32)])
    colp = jnp.concatenate([col, jnp.full((pad,), N, jnp.int32)])
    row3 = rowp.reshape(NW, STEPS, CH)
    col3 = colp.reshape(NW, STEPS, CH)

    x8 = jnp.pad(x, ((0, 0), (0, 1)))
    w1p = jnp.pad(W1, ((0, 1), (0, 0)))
    b1r = b1.reshape(1, 64)
    b2r = b2.reshape(1, 32)
    bcr = bc.reshape(1, 2)

    zeros1 = jnp.zeros((NACCD,), jnp.float32)
    zeros8 = jnp.zeros((N_ACC, 8), jnp.float32)
    zeros16 = jnp.zeros((N_ACC, 16), jnp.float32)
    ones = jnp.ones((CH,), jnp.float32)

    deg_part = _make_deg()(col3, ones, zeros1)
    deg3 = deg_part.reshape(NC, NACCD)[:, :N].reshape(NC, N, 1)

    grid = (N // _B,)
    dis, g1 = pl.pallas_call(
        _p1_body,
        grid=grid,
        in_specs=[_part_spec(1), _row_spec(8)],
        out_specs=[_row_spec(1), _row_spec(8)],
        out_shape=[jax.ShapeDtypeStruct((N, 1), jnp.float32),
                   jax.ShapeDtypeStruct((N, 8), jnp.float32)],
    )(deg3, x8)

    spmm8 = _make_spmm(8)
    spmm16 = _make_spmm(16)

    s1 = spmm8(g1, row3, col3, zeros8)[:, :N, :]

    g2a, g2b = pl.pallas_call(
        _p2_body,
        grid=grid,
        in_specs=[_part_spec(8), _row_spec(8), _row_spec(1),
                  _full_spec((8, 64)), _full_spec((1, 64)),
                  _full_spec((64, 32))],
        out_specs=[_row_spec(16), _row_spec(16)],
        out_shape=[jax.ShapeDtypeStruct((N, 16), jnp.float32),
                   jax.ShapeDtypeStruct((N, 16), jnp.float32)],
    )(s1, g1, dis, w1p, b1r, W2)

    s2a = spmm16(g2a, row3, col3, zeros16)[:, :N, :]
    s2b = spmm16(g2b, row3, col3, zeros16)[:, :N, :]

    out = pl.pallas_call(
        _p3_body,
        grid=grid,
        in_specs=[_part_spec(16), _part_spec(16), _row_spec(16),
                  _row_spec(16), _row_spec(1), _full_spec((1, 32)),
                  _full_spec((32, 2)), _full_spec((1, 2))],
        out_specs=_row_spec(2),
        out_shape=jax.ShapeDtypeStruct((N, 2), jnp.float32),
    )(s2a, s2b, g2a, g2b, dis, b2r, Wc, bcr)

    return out


# speedup vs baseline: 26.1628x; 1.4400x over previous
import functools

import jax
import jax.numpy as jnp
from jax import lax
from jax.experimental import pallas as pl
from jax.experimental.pallas import tpu as pltpu
from jax.experimental.pallas import tpu_sc as plsc

N = 100000
E = 1600000
NC = 2
NS = 16
NW = NC * NS
CH = 128
IB = 40
STEPS = 400
NBLK = STEPS // IB
E_PAD = NW * STEPS * CH
N_ACC = 100096
ACC_PT = N_ACC // NS
NACCD = 114688
ACC_PTD = NACCD // NS


NBUF = 4
GRP = IB // NBUF


def _pipelined_edge_loop(table_hbm, row_ix, col_ix, acc,
                         rowbuf, colbuf, gbufs, sems, nblk):
    for blk in range(nblk):
        pltpu.sync_copy(row_ix(blk), rowbuf)
        pltpu.sync_copy(col_ix(blk), colbuf)
        for b in range(NBUF):
            pltpu.async_copy(table_hbm.at[rowbuf.at[b]], gbufs[b], sems[b])

        def group(g, carry):
            base = g * NBUF
            for b in range(NBUF):
                j = base + b
                pltpu.make_async_copy(table_hbm.at[rowbuf.at[j]],
                                      gbufs[b], sems[b]).wait()
                pltpu.sync_copy(gbufs[b], acc.at[colbuf.at[j]], add=True)
                pltpu.async_copy(table_hbm.at[rowbuf.at[j + NBUF]],
                                 gbufs[b], sems[b])
            return carry

        lax.fori_loop(0, GRP - 1, group, 0)
        for b in range(NBUF):
            j = (GRP - 1) * NBUF + b
            pltpu.make_async_copy(table_hbm.at[rowbuf.at[j]],
                                  gbufs[b], sems[b]).wait()
            pltpu.sync_copy(gbufs[b], acc.at[colbuf.at[j]], add=True)


def _spmm_scratch(D):
    return [
        pltpu.VMEM((IB, CH), jnp.int32),
        pltpu.VMEM((IB, CH), jnp.int32),
    ] + [pltpu.VMEM((CH, D), jnp.float32) for _ in range(NBUF)] + [
        pltpu.VMEM_SHARED((N_ACC, D), jnp.float32),
    ] + [pltpu.SemaphoreType.DMA for _ in range(NBUF)]


def _make_spmm_split(D):
    mesh = plsc.VectorSubcoreMesh(core_axis_name="c", subcore_axis_name="s")

    @functools.partial(
        pl.kernel,
        out_type=jax.ShapeDtypeStruct((NC, N_ACC, D), jnp.float32),
        mesh=mesh,
        scratch_types=_spmm_scratch(D),
        compiler_params=pltpu.CompilerParams(use_tc_tiling_on_sc=False),
    )
    def spmm(table_hbm, row_hbm, col_hbm, zeros_hbm, out_hbm,
             rowbuf, colbuf, g0, g1, g2, g3, acc, s0, s1, s2, s3):
        c = lax.axis_index("c")
        s = lax.axis_index("s")
        w = c * NS + s
        pltpu.sync_copy(zeros_hbm.at[pl.ds(s * ACC_PT, ACC_PT)],
                        acc.at[pl.ds(s * ACC_PT, ACC_PT)])
        plsc.subcore_barrier()
        _pipelined_edge_loop(
            table_hbm,
            lambda blk: row_hbm.at[w, pl.ds(blk * IB, IB)],
            lambda blk: col_hbm.at[w, pl.ds(blk * IB, IB)],
            acc, rowbuf, colbuf, (g0, g1, g2, g3), (s0, s1, s2, s3), NBLK)
        plsc.subcore_barrier()
        pltpu.sync_copy(acc.at[pl.ds(s * ACC_PT, ACC_PT)],
                        out_hbm.at[c, pl.ds(s * ACC_PT, ACC_PT)])

    return spmm


def _make_spmm_chunked(D, steps2, nblk2):
    mesh = plsc.VectorSubcoreMesh(core_axis_name="c", subcore_axis_name="s")

    @functools.partial(
        pl.kernel,
        out_type=jax.ShapeDtypeStruct((NC, N_ACC, D), jnp.float32),
        mesh=mesh,
        scratch_types=_spmm_scratch(D),
        compiler_params=pltpu.CompilerParams(use_tc_tiling_on_sc=False),
    )
    def spmm(table_hbm, row_hbm, col_hbm, zeros_hbm, out_hbm,
             rowbuf, colbuf, g0, g1, g2, g3, acc, s0, s1, s2, s3):
        c = lax.axis_index("c")
        s = lax.axis_index("s")
        pltpu.sync_copy(zeros_hbm.at[pl.ds(s * ACC_PT, ACC_PT)],
                        acc.at[pl.ds(s * ACC_PT, ACC_PT)])
        plsc.subcore_barrier()
        _pipelined_edge_loop(
            table_hbm,
            lambda blk: row_hbm.at[c, s, pl.ds(blk * IB, IB)],
            lambda blk: col_hbm.at[s, pl.ds(blk * IB, IB)],
            acc, rowbuf, colbuf, (g0, g1, g2, g3), (s0, s1, s2, s3), nblk2)
        plsc.subcore_barrier()
        pltpu.sync_copy(acc.at[pl.ds(s * ACC_PT, ACC_PT)],
                        out_hbm.at[c, pl.ds(s * ACC_PT, ACC_PT)])

    return spmm


def _make_deg():
    mesh = plsc.VectorSubcoreMesh(core_axis_name="c", subcore_axis_name="s")

    @functools.partial(
        pl.kernel,
        out_type=jax.ShapeDtypeStruct((NC * NACCD,), jnp.float32),
        mesh=mesh,
        scratch_types=[
            pltpu.VMEM((IB, CH), jnp.int32),
            pltpu.VMEM((CH,), jnp.float32),
            pltpu.VMEM_SHARED((NACCD,), jnp.float32),
        ],
        compiler_params=pltpu.CompilerParams(use_tc_tiling_on_sc=False),
    )
    def deg(col_hbm, ones_hbm, zeros_hbm, out_hbm, colbuf, onesbuf, acc):
        c = lax.axis_index("c")
        s = lax.axis_index("s")
        w = c * NS + s
        pltpu.sync_copy(ones_hbm, onesbuf)
        pltpu.sync_copy(zeros_hbm.at[pl.ds(s * ACC_PTD, ACC_PTD)],
                        acc.at[pl.ds(s * ACC_PTD, ACC_PTD)])
        plsc.subcore_barrier()
        for blk in range(NBLK):
            pltpu.sync_copy(col_hbm.at[w, pl.ds(blk * IB, IB)], colbuf)

            def body(i, carry):
                pltpu.sync_copy(onesbuf, acc.at[colbuf.at[i]], add=True)
                return carry

            lax.fori_loop(0, IB, body, 0)
        plsc.subcore_barrier()
        pltpu.sync_copy(acc.at[pl.ds(s * ACC_PTD, ACC_PTD)],
                        out_hbm.at[pl.ds(c * NACCD + s * ACC_PTD, ACC_PTD)])

    return deg


_B = 1000


def _p1_body(deg_ref, x_ref, dis_ref, g1_ref):
    deg = deg_ref[0, :, :] + deg_ref[1, :, :] + 1.0
    dis = lax.rsqrt(deg)
    dis_ref[...] = dis
    g1_ref[...] = x_ref[...] * dis


def _p2_body(s1_ref, g1_ref, dis_ref, w1_ref, b1_ref, w2_ref, g2_ref):
    dis = dis_ref[...]
    t = (s1_ref[0, :, :] + s1_ref[1, :, :] + g1_ref[...]) * dis
    h1 = jnp.dot(t, w1_ref[...], preferred_element_type=jnp.float32)
    h1 = jnp.maximum(h1 + b1_ref[...], 0.0)
    g2 = jnp.dot(h1, w2_ref[...], preferred_element_type=jnp.float32) * dis
    g2_ref[0, :, :] = g2[:, :16]
    g2_ref[1, :, :] = g2[:, 16:]


def _p3_body(s2_ref, g2_ref, dis_ref, b2_ref, wc_ref, bc_ref, out_ref):
    dis = dis_ref[...]
    ua = s2_ref[0, :, :] + g2_ref[0, :, :]
    ub = s2_ref[1, :, :] + g2_ref[1, :, :]
    u = jnp.concatenate([ua, ub], axis=1) * dis
    h2 = jnp.maximum(u + b2_ref[...], 0.0)
    logits = jnp.dot(h2, wc_ref[...], preferred_element_type=jnp.float32)
    logits = logits + bc_ref[...]
    m = jnp.max(logits, axis=1, keepdims=True)
    lse = jnp.log(jnp.sum(jnp.exp(logits - m), axis=1, keepdims=True)) + m
    out_ref[...] = logits - lse


def _row_spec(d):
    return pl.BlockSpec((_B, d), lambda i: (i, 0))


def _full_spec(shape):
    nd = len(shape)
    return pl.BlockSpec(shape, lambda i: (0,) * nd)


def _part_spec(d):
    return pl.BlockSpec((NC, _B, d), lambda i: (0, i, 0))


def kernel(x, edge_index, W1, b1, W2, b2, Wc, bc):
    row = edge_index[0].astype(jnp.int32)
    col = edge_index[1].astype(jnp.int32)
    pad = E_PAD - E
    rowp = jnp.concatenate([row, jnp.zeros((pad,), jnp.int32)])
    colp = jnp.concatenate([col, jnp.full((pad,), N, jnp.int32)])
    row3 = rowp.reshape(NW, STEPS, CH)
    col3 = colp.reshape(NW, STEPS, CH)
    steps2 = E_PAD // (NS * CH)
    nblk2 = steps2 // IB
    row2 = jnp.stack([rowp, rowp + N]).reshape(NC, NS, steps2, CH)
    col2 = colp.reshape(NS, steps2, CH)

    x8 = jnp.pad(x, ((0, 0), (0, 1)))
    w1p = jnp.pad(W1, ((0, 1), (0, 0)))
    b1r = b1.reshape(1, 64)
    b2r = b2.reshape(1, 32)
    bcr = bc.reshape(1, 2)

    zeros1 = jnp.zeros((NACCD,), jnp.float32)
    zeros8 = jnp.zeros((N_ACC, 8), jnp.float32)
    zeros16 = jnp.zeros((N_ACC, 16), jnp.float32)
    ones = jnp.ones((CH,), jnp.float32)

    deg_part = _make_deg()(col3, ones, zeros1)
    deg3 = deg_part.reshape(NC, NACCD)[:, :N].reshape(NC, N, 1)

    grid = (N // _B,)
    dis, g1 = pl.pallas_call(
        _p1_body,
        grid=grid,
        in_specs=[_part_spec(1), _row_spec(8)],
        out_specs=[_row_spec(1), _row_spec(8)],
        out_shape=[jax.ShapeDtypeStruct((N, 1), jnp.float32),
                   jax.ShapeDtypeStruct((N, 8), jnp.float32)],
    )(deg3, x8)

    spmm8 = _make_spmm_split(8)
    spmm16 = _make_spmm_chunked(16, steps2, nblk2)

    s1 = spmm8(g1, row3, col3, zeros8)[:, :N, :]

    g2 = pl.pallas_call(
        _p2_body,
        grid=grid,
        in_specs=[_part_spec(8), _row_spec(8), _row_spec(1),
                  _full_spec((8, 64)), _full_spec((1, 64)),
                  _full_spec((64, 32))],
        out_specs=_part_spec(16),
        out_shape=jax.ShapeDtypeStruct((NC, N, 16), jnp.float32),
    )(s1, g1, dis, w1p, b1r, W2)

    gf = g2.reshape(NC * N, 16)
    s2 = spmm16(gf, row2, col2, zeros16)[:, :N, :]

    out = pl.pallas_call(
        _p3_body,
        grid=grid,
        in_specs=[_part_spec(16), _part_spec(16), _row_spec(1),
                  _full_spec((1, 32)), _full_spec((32, 2)),
                  _full_spec((1, 2))],
        out_specs=_row_spec(2),
        out_shape=jax.ShapeDtypeStruct((N, 2), jnp.float32),
    )(s2, g2, dis, b2r, Wc, bcr)

    return out
